# baseline (speedup 1.0000x reference)
import functools

import jax
import jax.numpy as jnp
from jax import lax
from jax.experimental import pallas as pl
from jax.experimental.pallas import tpu as pltpu
from jax.experimental.pallas import tpu_sc as plsc

N = 10000
E = 320000
D = 128

NC = 2
NS = 16
NW = NC * NS
L = 16
FPT = D // NW
N4 = N * FPT
EPW = E // NW
CH = 16000
NCH = E // CH

_mesh = plsc.VectorSubcoreMesh(
    core_axis_name="c", subcore_axis_name="s", num_cores=NC, num_subcores=NS)
_sc_params = pltpu.CompilerParams(needs_layout_passes=False)


def _wid():
    return lax.axis_index("s") * NC + lax.axis_index("c")


@functools.partial(
    pl.kernel,
    out_type=jax.ShapeDtypeStruct((NW, N), jnp.float32),
    mesh=_mesh,
    compiler_params=_sc_params,
    scratch_types=[
        pltpu.VMEM((N,), jnp.float32),
        pltpu.VMEM((EPW,), jnp.int32),
        pltpu.VMEM((EPW,), jnp.float32),
        pltpu.SemaphoreType.DMA,
    ],
)
def _deg_sc(col_hbm, w_hbm, out_hbm, acc_v, col_v, w_v, sem):
    wid = _wid()
    base = wid * EPW
    cps = [pltpu.make_async_copy(col_hbm.at[pl.ds(base, EPW)], col_v, sem),
           pltpu.make_async_copy(w_hbm.at[pl.ds(base, EPW)], w_v, sem)]
    for cp in cps:
        cp.start()

    @plsc.parallel_loop(0, N // L, unroll=8)
    def _(i):
        acc_v[pl.ds(i * L, L)] = jnp.zeros((L,), jnp.float32)

    for cp in cps:
        cp.wait()

    @plsc.parallel_loop(0, EPW // L, unroll=8)
    def _(g):
        s = pl.ds(g * L, L)
        plsc.addupdate_scatter(acc_v, [col_v[s]], w_v[s])

    pltpu.sync_copy(acc_v, out_hbm.at[wid])


@functools.partial(
    pl.kernel,
    out_type=jax.ShapeDtypeStruct((E,), jnp.float32),
    mesh=_mesh,
    compiler_params=_sc_params,
    scratch_types=[
        pltpu.VMEM((N,), jnp.float32),
        pltpu.VMEM((EPW,), jnp.int32),
        pltpu.VMEM((EPW,), jnp.int32),
        pltpu.VMEM((EPW,), jnp.float32),
        pltpu.VMEM((EPW,), jnp.float32),
        pltpu.SemaphoreType.DMA,
    ],
)
def _norm_sc(row_hbm, col_hbm, w_hbm, dinv_hbm, out_hbm,
             dinv_v, row_v, col_v, w_v, nrm_v, sem):
    wid = _wid()
    base = wid * EPW
    cps = [pltpu.make_async_copy(dinv_hbm, dinv_v, sem),
           pltpu.make_async_copy(row_hbm.at[pl.ds(base, EPW)], row_v, sem),
           pltpu.make_async_copy(col_hbm.at[pl.ds(base, EPW)], col_v, sem),
           pltpu.make_async_copy(w_hbm.at[pl.ds(base, EPW)], w_v, sem)]
    for cp in cps:
        cp.start()
    for cp in cps:
        cp.wait()

    @plsc.parallel_loop(0, EPW // L, unroll=8)
    def _(g):
        s = pl.ds(g * L, L)
        dr = plsc.load_gather(dinv_v, [row_v[s]])
        dc = plsc.load_gather(dinv_v, [col_v[s]])
        nrm_v[s] = dr * w_v[s] * dc

    pltpu.sync_copy(nrm_v, out_hbm.at[pl.ds(base, EPW)])


@functools.partial(
    pl.kernel,
    out_type=jax.ShapeDtypeStruct((NW * FPT, N), jnp.float32),
    mesh=_mesh,
    compiler_params=_sc_params,
    scratch_types=[
        pltpu.VMEM((N,), jnp.int32),
        pltpu.VMEM((N,), jnp.int32),
        pltpu.VMEM((N,), jnp.float32),
        pltpu.VMEM((N,), jnp.float32),
        pltpu.VMEM((N,), jnp.float32),
        pltpu.VMEM((N,), jnp.float32),
        pltpu.VMEM((CH,), jnp.int32),
        pltpu.VMEM((CH,), jnp.float32),
        pltpu.VMEM((CH,), jnp.int32),
        pltpu.VMEM((CH,), jnp.float32),
        pltpu.SemaphoreType.DMA,
        pltpu.SemaphoreType.DMA,
        pltpu.SemaphoreType.DMA,
    ],
)
def _agg_sc(xw_hbm, pc_hbm, nrm_hbm, out_hbm,
            xw0_v, xw1_v, acc0_v, acc1_v, acc2_v, acc3_v,
            pc_a, nrm_a, pc_b, nrm_b, sem_a, sem_b, sem_s):
    wid = _wid()
    slabs = (xw0_v, xw1_v)
    accs = (acc0_v, acc1_v, acc2_v, acc3_v)
    bufs = ((pc_a, nrm_a, sem_a), (pc_b, nrm_b, sem_b))

    slab_cp = [
        pltpu.make_async_copy(xw_hbm.at[wid * (FPT // 2) + j], slabs[j], sem_s)
        for j in range(FPT // 2)]
    for cp in slab_cp:
        cp.start()

    @plsc.parallel_loop(0, N // L, unroll=8)
    def _(i):
        z = jnp.zeros((L,), jnp.float32)
        for a in accs:
            a[pl.ds(i * L, L)] = z

    for cp in slab_cp:
        cp.wait()

    def start(ci, buf):
        pcv, nrmv, sem = buf
        pltpu.make_async_copy(pc_hbm.at[pl.ds(ci * CH, CH)], pcv, sem).start()
        pltpu.make_async_copy(nrm_hbm.at[pl.ds(ci * CH, CH)], nrmv, sem).start()

    def wait(buf):
        pcv, nrmv, sem = buf
        pltpu.make_async_copy(pc_hbm.at[pl.ds(0, CH)], pcv, sem).wait()
        pltpu.make_async_copy(nrm_hbm.at[pl.ds(0, CH)], nrmv, sem).wait()

    def process(buf):
        pcv, nrmv, _ = buf

        @plsc.parallel_loop(0, CH // L, unroll=16)
        def _(g):
            s = pl.ds(g * L, L)
            pc16 = pcv[s]
            n16 = nrmv[s]
            r16 = lax.shift_right_logical(pc16, 16)
            c16 = pc16 & 0xFFFF
            for j in range(FPT // 2):
                w = plsc.load_gather(slabs[j], [r16])
                lo, hi = plsc.unpack(
                    plsc.bitcast(w, jnp.bfloat16),
                    format=plsc.PackFormat.INTERLEAVED,
                    preferred_element_type=jnp.float32)
                plsc.addupdate_scatter(accs[2 * j], [c16], lo * n16)
                plsc.addupdate_scatter(accs[2 * j + 1], [c16], hi * n16)

    start(0, bufs[0])
    for ci in range(NCH):
        if ci + 1 < NCH:
            start(ci + 1, bufs[(ci + 1) % 2])
        cur = bufs[ci % 2]
        wait(cur)
        process(cur)

    out_cp = [
        pltpu.make_async_copy(accs[k], out_hbm.at[wid * FPT + k], sem_s)
        for k in range(FPT)]
    for cp in out_cp:
        cp.start()
    for cp in out_cp:
        cp.wait()


def _gru(w0, wih, whh, bih, bhh):
    dn = (((1,), (1,)), ((), ()))
    gi = lax.dot_general(w0, wih, dn, preferred_element_type=jnp.float32,
                         precision=lax.Precision.HIGHEST) + bih
    gh = lax.dot_general(w0, whh, dn, preferred_element_type=jnp.float32,
                         precision=lax.Precision.HIGHEST) + bhh
    ir, iz, inn = jnp.split(gi, 3, axis=-1)
    hr, hz, hn = jnp.split(gh, 3, axis=-1)
    r = jax.nn.sigmoid(ir + hr)
    z = jax.nn.sigmoid(iz + hz)
    n = jnp.tanh(inn + r * hn)
    return (1.0 - z) * n + z * w0


def _mm(a, b):
    return lax.dot_general(a, b, (((1,), (0,)), ((), ())),
                           preferred_element_type=jnp.float32,
                           precision=lax.Precision.HIGHEST)


def _mmT(w, a):
    return lax.dot_general(w, a, (((0,), (0,)), ((), ())),
                           preferred_element_type=jnp.float32,
                           precision=lax.Precision.HIGHEST)


def _pack_pairs(xt):
    pairs = xt.reshape(D // 2, 2, N)
    lo = lax.bitcast_convert_type(
        pairs[:, 0, :].astype(jnp.bfloat16), jnp.uint16).astype(jnp.uint32)
    hi = lax.bitcast_convert_type(
        pairs[:, 1, :].astype(jnp.bfloat16), jnp.uint16).astype(jnp.uint32)
    return lax.bitcast_convert_type(lo | (hi << 16), jnp.int32)


def _prep_tc(x_ref, iw1_ref, wih1_ref, whh1_ref, bih1_ref, bhh1_ref,
             iw2_ref, wih2_ref, whh2_ref, bih2_ref, bhh2_ref,
             row_ref, col_ref, w2_ref, xw1t_ref, p1_ref, pc_ref):
    w1 = _gru(iw1_ref[...], wih1_ref[...], whh1_ref[...], bih1_ref[...],
              bhh1_ref[...])
    w2_ref[...] = _gru(iw2_ref[...], wih2_ref[...], whh2_ref[...],
                       bih2_ref[...], bhh2_ref[...])
    xw1t = lax.dot_general(w1, x_ref[...], (((0,), (1,)), ((), ())),
                           preferred_element_type=jnp.float32,
                           precision=lax.Precision.HIGHEST)
    xw1t_ref[...] = xw1t
    p1_ref[...] = _pack_pairs(xw1t)
    r = row_ref[...].astype(jnp.uint32)
    c = col_ref[...].astype(jnp.uint32)
    pc_ref[...] = lax.bitcast_convert_type((r << 16) | c, jnp.int32)


def _dinv_tc(degp_ref, dinv_ref, dinv2_ref):
    deg = jnp.sum(degp_ref[...], axis=0, keepdims=True) + 1.0
    dinv = jnp.where(deg > 0, lax.rsqrt(jnp.maximum(deg, 1e-12)), 0.0)
    dinv_ref[...] = dinv
    dinv2_ref[...] = dinv * dinv


def _comb_mm_tc(e1_ref, xwt_ref, d2_ref, w_ref, xw2t_ref, p2_ref):
    h = jnp.maximum(e1_ref[...] + d2_ref[...] * xwt_ref[...], 0.0)
    xw2t = _mmT(w_ref[...], h)
    xw2t_ref[...] = xw2t
    p2_ref[...] = _pack_pairs(xw2t)


def _final_tc(e2_ref, xwt_ref, d2_ref, out_ref):
    out_ref[...] = jnp.transpose(
        jnp.maximum(e2_ref[...] + d2_ref[...] * xwt_ref[...], 0.0), (1, 0))


def kernel(x, edge_index, edge_weight, init_w1, wih1, whh1, bih1, bhh1,
           init_w2, wih2, whh2, bih2, bhh2):
    row = edge_index[0]
    col = edge_index[1]

    f32 = jnp.float32
    w2, xw1t, p1, pc2d = pl.pallas_call(
        _prep_tc,
        out_shape=(jax.ShapeDtypeStruct((D, D), f32),
                   jax.ShapeDtypeStruct((D, N), f32),
                   jax.ShapeDtypeStruct((D // 2, N), jnp.int32),
                   jax.ShapeDtypeStruct((E // D, D), jnp.int32)),
    )(x, init_w1, wih1, whh1, bih1.reshape(1, 3 * D), bhh1.reshape(1, 3 * D),
      init_w2, wih2, whh2, bih2.reshape(1, 3 * D), bhh2.reshape(1, 3 * D),
      row.reshape(E // D, D), col.reshape(E // D, D))
    pc = pc2d.reshape(E)

    deg_p = _deg_sc(col, edge_weight)

    dinv, dinv2 = pl.pallas_call(
        _dinv_tc,
        out_shape=(jax.ShapeDtypeStruct((1, N), f32),
                   jax.ShapeDtypeStruct((1, N), f32)),
    )(deg_p)

    nrm = _norm_sc(row, col, edge_weight, dinv.reshape(N))

    e1p = _agg_sc(p1, pc, nrm)

    xw2t, p2 = pl.pallas_call(
        _comb_mm_tc,
        out_shape=(jax.ShapeDtypeStruct((D, N), f32),
                   jax.ShapeDtypeStruct((D // 2, N), jnp.int32)),
    )(e1p, xw1t, dinv2, w2)

    e2p = _agg_sc(p2, pc, nrm)

    out = pl.pallas_call(
        _final_tc,
        out_shape=jax.ShapeDtypeStruct((N, D), f32),
    )(e2p, xw2t, dinv2)
    return out


# speedup vs baseline: 1.0484x; 1.0484x over previous
import functools

import jax
import jax.numpy as jnp
from jax import lax
from jax.experimental import pallas as pl
from jax.experimental.pallas import tpu as pltpu
from jax.experimental.pallas import tpu_sc as plsc

N = 10000
E = 320000
D = 128

NC = 2
NS = 16
NW = NC * NS
L = 16
FPT = D // NW
N4 = N * FPT
EPW = E // NW
CH = 16000
NCH = E // CH

_mesh = plsc.VectorSubcoreMesh(
    core_axis_name="c", subcore_axis_name="s", num_cores=NC, num_subcores=NS)
_sc_params = pltpu.CompilerParams(needs_layout_passes=False)


def _wid():
    return lax.axis_index("s") * NC + lax.axis_index("c")


@functools.partial(
    pl.kernel,
    out_type=jax.ShapeDtypeStruct((NW, N), jnp.float32),
    mesh=_mesh,
    compiler_params=_sc_params,
    scratch_types=[
        pltpu.VMEM((N,), jnp.float32),
        pltpu.VMEM((EPW,), jnp.int32),
        pltpu.VMEM((EPW,), jnp.float32),
        pltpu.SemaphoreType.DMA,
    ],
)
def _deg_sc(col_hbm, w_hbm, out_hbm, acc_v, col_v, w_v, sem):
    wid = _wid()
    base = wid * EPW
    cps = [pltpu.make_async_copy(col_hbm.at[pl.ds(base, EPW)], col_v, sem),
           pltpu.make_async_copy(w_hbm.at[pl.ds(base, EPW)], w_v, sem)]
    for cp in cps:
        cp.start()

    @plsc.parallel_loop(0, N // L, unroll=8)
    def _(i):
        acc_v[pl.ds(i * L, L)] = jnp.zeros((L,), jnp.float32)

    for cp in cps:
        cp.wait()

    @plsc.parallel_loop(0, EPW // L, unroll=8)
    def _(g):
        s = pl.ds(g * L, L)
        plsc.addupdate_scatter(acc_v, [col_v[s]], w_v[s])

    pltpu.sync_copy(acc_v, out_hbm.at[wid])


@functools.partial(
    pl.kernel,
    out_type=jax.ShapeDtypeStruct((E,), jnp.float32),
    mesh=_mesh,
    compiler_params=_sc_params,
    scratch_types=[
        pltpu.VMEM((N,), jnp.float32),
        pltpu.VMEM((EPW,), jnp.int32),
        pltpu.VMEM((EPW,), jnp.int32),
        pltpu.VMEM((EPW,), jnp.float32),
        pltpu.VMEM((EPW,), jnp.float32),
        pltpu.SemaphoreType.DMA,
    ],
)
def _norm_sc(row_hbm, col_hbm, w_hbm, dinv_hbm, out_hbm,
             dinv_v, row_v, col_v, w_v, nrm_v, sem):
    wid = _wid()
    base = wid * EPW
    cps = [pltpu.make_async_copy(dinv_hbm, dinv_v, sem),
           pltpu.make_async_copy(row_hbm.at[pl.ds(base, EPW)], row_v, sem),
           pltpu.make_async_copy(col_hbm.at[pl.ds(base, EPW)], col_v, sem),
           pltpu.make_async_copy(w_hbm.at[pl.ds(base, EPW)], w_v, sem)]
    for cp in cps:
        cp.start()
    for cp in cps:
        cp.wait()

    @plsc.parallel_loop(0, EPW // L, unroll=8)
    def _(g):
        s = pl.ds(g * L, L)
        dr = plsc.load_gather(dinv_v, [row_v[s]])
        dc = plsc.load_gather(dinv_v, [col_v[s]])
        nrm_v[s] = dr * w_v[s] * dc

    pltpu.sync_copy(nrm_v, out_hbm.at[pl.ds(base, EPW)])


@functools.partial(
    pl.kernel,
    out_type=jax.ShapeDtypeStruct((NW * FPT, N), jnp.float32),
    mesh=_mesh,
    compiler_params=_sc_params,
    scratch_types=[
        pltpu.VMEM((N,), jnp.int32),
        pltpu.VMEM((N,), jnp.int32),
        pltpu.VMEM((N,), jnp.float32),
        pltpu.VMEM((N,), jnp.float32),
        pltpu.VMEM((N,), jnp.float32),
        pltpu.VMEM((N,), jnp.float32),
        pltpu.VMEM((CH,), jnp.int32),
        pltpu.VMEM((CH,), jnp.float32),
        pltpu.VMEM((CH,), jnp.int32),
        pltpu.VMEM((CH,), jnp.float32),
        pltpu.SemaphoreType.DMA,
        pltpu.SemaphoreType.DMA,
        pltpu.SemaphoreType.DMA,
    ],
)
def _agg_sc(xw_hbm, pc_hbm, nrm_hbm, out_hbm,
            xw0_v, xw1_v, acc0_v, acc1_v, acc2_v, acc3_v,
            pc_a, nrm_a, pc_b, nrm_b, sem_a, sem_b, sem_s):
    wid = _wid()
    slabs = (xw0_v, xw1_v)
    accs = (acc0_v, acc1_v, acc2_v, acc3_v)
    bufs = ((pc_a, nrm_a, sem_a), (pc_b, nrm_b, sem_b))

    slab_cp = [
        pltpu.make_async_copy(xw_hbm.at[wid * (FPT // 2) + j], slabs[j], sem_s)
        for j in range(FPT // 2)]
    for cp in slab_cp:
        cp.start()

    @plsc.parallel_loop(0, N // L, unroll=8)
    def _(i):
        z = jnp.zeros((L,), jnp.float32)
        for a in accs:
            a[pl.ds(i * L, L)] = z

    for cp in slab_cp:
        cp.wait()

    def start(ci, buf):
        pcv, nrmv, sem = buf
        pltpu.make_async_copy(pc_hbm.at[pl.ds(ci * CH, CH)], pcv, sem).start()
        pltpu.make_async_copy(nrm_hbm.at[pl.ds(ci * CH, CH)], nrmv, sem).start()

    def wait(buf):
        pcv, nrmv, sem = buf
        pltpu.make_async_copy(pc_hbm.at[pl.ds(0, CH)], pcv, sem).wait()
        pltpu.make_async_copy(nrm_hbm.at[pl.ds(0, CH)], nrmv, sem).wait()

    def process(buf):
        pcv, nrmv, _ = buf

        @plsc.parallel_loop(0, CH // L, unroll=8)
        def _(g):
            s = pl.ds(g * L, L)
            pc16 = pcv[s]
            n16 = nrmv[s]
            r16 = lax.shift_right_logical(pc16, 16)
            c16 = pc16 & 0xFFFF
            for j in range(FPT // 2):
                w = plsc.load_gather(slabs[j], [r16])
                lo, hi = plsc.unpack(
                    plsc.bitcast(w, jnp.bfloat16),
                    format=plsc.PackFormat.INTERLEAVED,
                    preferred_element_type=jnp.float32)
                plsc.addupdate_scatter(accs[2 * j], [c16], lo * n16)
                plsc.addupdate_scatter(accs[2 * j + 1], [c16], hi * n16)

    start(0, bufs[0])
    for ci in range(NCH):
        if ci + 1 < NCH:
            start(ci + 1, bufs[(ci + 1) % 2])
        cur = bufs[ci % 2]
        wait(cur)
        process(cur)

    out_cp = [
        pltpu.make_async_copy(accs[k], out_hbm.at[wid * FPT + k], sem_s)
        for k in range(FPT)]
    for cp in out_cp:
        cp.start()
    for cp in out_cp:
        cp.wait()


def _gru(w0, wih, whh, bih, bhh):
    dn = (((1,), (1,)), ((), ()))
    gi = lax.dot_general(w0, wih, dn, preferred_element_type=jnp.float32,
                         precision=lax.Precision.HIGHEST) + bih
    gh = lax.dot_general(w0, whh, dn, preferred_element_type=jnp.float32,
                         precision=lax.Precision.HIGHEST) + bhh
    ir, iz, inn = jnp.split(gi, 3, axis=-1)
    hr, hz, hn = jnp.split(gh, 3, axis=-1)
    r = jax.nn.sigmoid(ir + hr)
    z = jax.nn.sigmoid(iz + hz)
    n = jnp.tanh(inn + r * hn)
    return (1.0 - z) * n + z * w0


def _mm(a, b):
    return lax.dot_general(a, b, (((1,), (0,)), ((), ())),
                           preferred_element_type=jnp.float32,
                           precision=lax.Precision.HIGHEST)


def _mmT(w, a):
    return lax.dot_general(w, a, (((0,), (0,)), ((), ())),
                           preferred_element_type=jnp.float32,
                           precision=lax.Precision.HIGHEST)


def _pack_pairs(xt):
    pairs = xt.reshape(D // 2, 2, N)
    lo = lax.bitcast_convert_type(
        pairs[:, 0, :].astype(jnp.bfloat16), jnp.uint16).astype(jnp.uint32)
    hi = lax.bitcast_convert_type(
        pairs[:, 1, :].astype(jnp.bfloat16), jnp.uint16).astype(jnp.uint32)
    return lax.bitcast_convert_type(lo | (hi << 16), jnp.int32)


def _prep_tc(x_ref, iw1_ref, wih1_ref, whh1_ref, bih1_ref, bhh1_ref,
             iw2_ref, wih2_ref, whh2_ref, bih2_ref, bhh2_ref,
             row_ref, col_ref, w2_ref, xw1t_ref, p1_ref, pc_ref):
    w1 = _gru(iw1_ref[...], wih1_ref[...], whh1_ref[...], bih1_ref[...],
              bhh1_ref[...])
    w2_ref[...] = _gru(iw2_ref[...], wih2_ref[...], whh2_ref[...],
                       bih2_ref[...], bhh2_ref[...])
    xw1t = lax.dot_general(w1, x_ref[...], (((0,), (1,)), ((), ())),
                           preferred_element_type=jnp.float32,
                           precision=lax.Precision.HIGHEST)
    xw1t_ref[...] = xw1t
    p1_ref[...] = _pack_pairs(xw1t)
    r = row_ref[...].astype(jnp.uint32)
    c = col_ref[...].astype(jnp.uint32)
    pc_ref[...] = lax.bitcast_convert_type((r << 16) | c, jnp.int32)


def _dinv_tc(degp_ref, dinv_ref, dinv2_ref):
    deg = jnp.sum(degp_ref[...], axis=0, keepdims=True) + 1.0
    dinv = jnp.where(deg > 0, lax.rsqrt(jnp.maximum(deg, 1e-12)), 0.0)
    dinv_ref[...] = dinv
    dinv2_ref[...] = dinv * dinv


def _comb_mm_tc(e1_ref, xwt_ref, d2_ref, w_ref, xw2t_ref, p2_ref):
    h = jnp.maximum(e1_ref[...] + d2_ref[...] * xwt_ref[...], 0.0)
    xw2t = _mmT(w_ref[...], h)
    xw2t_ref[...] = xw2t
    p2_ref[...] = _pack_pairs(xw2t)


def _final_tc(e2_ref, xwt_ref, d2_ref, out_ref):
    out_ref[...] = jnp.transpose(
        jnp.maximum(e2_ref[...] + d2_ref[...] * xwt_ref[...], 0.0), (1, 0))


def kernel(x, edge_index, edge_weight, init_w1, wih1, whh1, bih1, bhh1,
           init_w2, wih2, whh2, bih2, bhh2):
    row = edge_index[0]
    col = edge_index[1]

    f32 = jnp.float32
    w2, xw1t, p1, pc2d = pl.pallas_call(
        _prep_tc,
        out_shape=(jax.ShapeDtypeStruct((D, D), f32),
                   jax.ShapeDtypeStruct((D, N), f32),
                   jax.ShapeDtypeStruct((D // 2, N), jnp.int32),
                   jax.ShapeDtypeStruct((E // D, D), jnp.int32)),
    )(x, init_w1, wih1, whh1, bih1.reshape(1, 3 * D), bhh1.reshape(1, 3 * D),
      init_w2, wih2, whh2, bih2.reshape(1, 3 * D), bhh2.reshape(1, 3 * D),
      row.reshape(E // D, D), col.reshape(E // D, D))
    pc = pc2d.reshape(E)

    deg_p = _deg_sc(col, edge_weight)

    dinv, dinv2 = pl.pallas_call(
        _dinv_tc,
        out_shape=(jax.ShapeDtypeStruct((1, N), f32),
                   jax.ShapeDtypeStruct((1, N), f32)),
    )(deg_p)

    nrm = _norm_sc(row, col, edge_weight, dinv.reshape(N))

    e1p = _agg_sc(p1, pc, nrm)

    xw2t, p2 = pl.pallas_call(
        _comb_mm_tc,
        out_shape=(jax.ShapeDtypeStruct((D, N), f32),
                   jax.ShapeDtypeStruct((D // 2, N), jnp.int32)),
    )(e1p, xw1t, dinv2, w2)

    e2p = _agg_sc(p2, pc, nrm)

    out = pl.pallas_call(
        _final_tc,
        out_shape=jax.ShapeDtypeStruct((N, D), f32),
    )(e2p, xw2t, dinv2)
    return out


# speedup vs baseline: 1.0535x; 1.0048x over previous
import functools

import jax
import jax.numpy as jnp
from jax import lax
from jax.experimental import pallas as pl
from jax.experimental.pallas import tpu as pltpu
from jax.experimental.pallas import tpu_sc as plsc

N = 10000
E = 320000
D = 128

NC = 2
NS = 16
NW = NC * NS
L = 16
FPT = D // NW
N4 = N * FPT
EPW = E // NW
CH = 16000
NCH = E // CH

_mesh = plsc.VectorSubcoreMesh(
    core_axis_name="c", subcore_axis_name="s", num_cores=NC, num_subcores=NS)
_sc_params = pltpu.CompilerParams(needs_layout_passes=False)


def _wid():
    return lax.axis_index("s") * NC + lax.axis_index("c")


@functools.partial(
    pl.kernel,
    out_type=jax.ShapeDtypeStruct((NW, N), jnp.float32),
    mesh=_mesh,
    compiler_params=_sc_params,
    scratch_types=[
        pltpu.VMEM((N,), jnp.float32),
        pltpu.VMEM((EPW,), jnp.int32),
        pltpu.VMEM((EPW,), jnp.float32),
        pltpu.SemaphoreType.DMA,
    ],
)
def _deg_sc(col_hbm, w_hbm, out_hbm, acc_v, col_v, w_v, sem):
    wid = _wid()
    base = wid * EPW
    cps = [pltpu.make_async_copy(col_hbm.at[pl.ds(base, EPW)], col_v, sem),
           pltpu.make_async_copy(w_hbm.at[pl.ds(base, EPW)], w_v, sem)]
    for cp in cps:
        cp.start()

    @plsc.parallel_loop(0, N // L, unroll=8)
    def _(i):
        acc_v[pl.ds(i * L, L)] = jnp.zeros((L,), jnp.float32)

    for cp in cps:
        cp.wait()

    @plsc.parallel_loop(0, EPW // L, unroll=8)
    def _(g):
        s = pl.ds(g * L, L)
        plsc.addupdate_scatter(acc_v, [col_v[s]], w_v[s])

    pltpu.sync_copy(acc_v, out_hbm.at[wid])


@functools.partial(
    pl.kernel,
    out_type=jax.ShapeDtypeStruct((E,), jnp.float32),
    mesh=_mesh,
    compiler_params=_sc_params,
    scratch_types=[
        pltpu.VMEM((N,), jnp.float32),
        pltpu.VMEM((EPW,), jnp.int32),
        pltpu.VMEM((EPW,), jnp.int32),
        pltpu.VMEM((EPW,), jnp.float32),
        pltpu.VMEM((EPW,), jnp.float32),
        pltpu.SemaphoreType.DMA,
    ],
)
def _norm_sc(row_hbm, col_hbm, w_hbm, dinv_hbm, out_hbm,
             dinv_v, row_v, col_v, w_v, nrm_v, sem):
    wid = _wid()
    base = wid * EPW
    cps = [pltpu.make_async_copy(dinv_hbm, dinv_v, sem),
           pltpu.make_async_copy(row_hbm.at[pl.ds(base, EPW)], row_v, sem),
           pltpu.make_async_copy(col_hbm.at[pl.ds(base, EPW)], col_v, sem),
           pltpu.make_async_copy(w_hbm.at[pl.ds(base, EPW)], w_v, sem)]
    for cp in cps:
        cp.start()
    for cp in cps:
        cp.wait()

    @plsc.parallel_loop(0, EPW // L, unroll=8)
    def _(g):
        s = pl.ds(g * L, L)
        dr = plsc.load_gather(dinv_v, [row_v[s]])
        dc = plsc.load_gather(dinv_v, [col_v[s]])
        nrm_v[s] = dr * w_v[s] * dc

    pltpu.sync_copy(nrm_v, out_hbm.at[pl.ds(base, EPW)])


@functools.partial(
    pl.kernel,
    out_type=jax.ShapeDtypeStruct((NW * FPT, N), jnp.float32),
    mesh=_mesh,
    compiler_params=_sc_params,
    scratch_types=[
        pltpu.VMEM((N,), jnp.int32),
        pltpu.VMEM((N,), jnp.int32),
        pltpu.VMEM((N,), jnp.float32),
        pltpu.VMEM((N,), jnp.float32),
        pltpu.VMEM((N,), jnp.float32),
        pltpu.VMEM((N,), jnp.float32),
        pltpu.VMEM((CH,), jnp.int32),
        pltpu.VMEM((CH,), jnp.float32),
        pltpu.VMEM((CH,), jnp.int32),
        pltpu.VMEM((CH,), jnp.float32),
        pltpu.SemaphoreType.DMA,
        pltpu.SemaphoreType.DMA,
        pltpu.SemaphoreType.DMA,
    ],
)
def _agg_sc(xw_hbm, pc_hbm, nrm_hbm, out_hbm,
            xw0_v, xw1_v, acc0_v, acc1_v, acc2_v, acc3_v,
            pc_a, nrm_a, pc_b, nrm_b, sem_a, sem_b, sem_s):
    wid = _wid()
    slabs = (xw0_v, xw1_v)
    accs = (acc0_v, acc1_v, acc2_v, acc3_v)
    bufs = ((pc_a, nrm_a, sem_a), (pc_b, nrm_b, sem_b))

    slab_cp = [
        pltpu.make_async_copy(xw_hbm.at[wid * (FPT // 2) + j], slabs[j], sem_s)
        for j in range(FPT // 2)]
    for cp in slab_cp:
        cp.start()

    @plsc.parallel_loop(0, N // L, unroll=8)
    def _(i):
        z = jnp.zeros((L,), jnp.float32)
        for a in accs:
            a[pl.ds(i * L, L)] = z

    for cp in slab_cp:
        cp.wait()

    def start(ci, buf):
        pcv, nrmv, sem = buf
        pltpu.make_async_copy(pc_hbm.at[pl.ds(ci * CH, CH)], pcv, sem).start()
        pltpu.make_async_copy(nrm_hbm.at[pl.ds(ci * CH, CH)], nrmv, sem).start()

    def wait(buf):
        pcv, nrmv, sem = buf
        pltpu.make_async_copy(pc_hbm.at[pl.ds(0, CH)], pcv, sem).wait()
        pltpu.make_async_copy(nrm_hbm.at[pl.ds(0, CH)], nrmv, sem).wait()

    def process(buf):
        pcv, nrmv, _ = buf

        @plsc.parallel_loop(0, CH // L, unroll=8)
        def _(g):
            s = pl.ds(g * L, L)
            pc16 = pcv[s]
            n16 = nrmv[s]
            r16 = lax.shift_right_logical(pc16, 16)
            c16 = pc16 & 0xFFFF
            for j in range(FPT // 2):
                w = plsc.load_gather(slabs[j], [r16])
                lo, hi = plsc.unpack(
                    plsc.bitcast(w, jnp.bfloat16),
                    format=plsc.PackFormat.INTERLEAVED,
                    preferred_element_type=jnp.float32)
                plsc.addupdate_scatter(accs[2 * j], [c16], lo * n16)
                plsc.addupdate_scatter(accs[2 * j + 1], [c16], hi * n16)

    start(0, bufs[0])
    for ci in range(NCH):
        if ci + 1 < NCH:
            start(ci + 1, bufs[(ci + 1) % 2])
        cur = bufs[ci % 2]
        wait(cur)
        process(cur)

    out_cp = [
        pltpu.make_async_copy(accs[k], out_hbm.at[wid * FPT + k], sem_s)
        for k in range(FPT)]
    for cp in out_cp:
        cp.start()
    for cp in out_cp:
        cp.wait()


def _gru(w0, wih, whh, bih, bhh):
    dn = (((1,), (1,)), ((), ()))
    gi = lax.dot_general(w0, wih, dn, preferred_element_type=jnp.float32,
                         precision=lax.Precision.HIGHEST) + bih
    gh = lax.dot_general(w0, whh, dn, preferred_element_type=jnp.float32,
                         precision=lax.Precision.HIGHEST) + bhh
    ir, iz, inn = jnp.split(gi, 3, axis=-1)
    hr, hz, hn = jnp.split(gh, 3, axis=-1)
    r = jax.nn.sigmoid(ir + hr)
    z = jax.nn.sigmoid(iz + hz)
    n = jnp.tanh(inn + r * hn)
    return (1.0 - z) * n + z * w0


def _mm(a, b):
    return lax.dot_general(a, b, (((1,), (0,)), ((), ())),
                           preferred_element_type=jnp.float32,
                           precision=lax.Precision.HIGHEST)


def _mmT(w, a):
    return lax.dot_general(w, a, (((0,), (0,)), ((), ())),
                           preferred_element_type=jnp.float32)


def _pack_pairs(xt):
    pairs = xt.reshape(D // 2, 2, N)
    lo = lax.bitcast_convert_type(
        pairs[:, 0, :].astype(jnp.bfloat16), jnp.uint16).astype(jnp.uint32)
    hi = lax.bitcast_convert_type(
        pairs[:, 1, :].astype(jnp.bfloat16), jnp.uint16).astype(jnp.uint32)
    return lax.bitcast_convert_type(lo | (hi << 16), jnp.int32)


def _prep_tc(x_ref, iw1_ref, wih1_ref, whh1_ref, bih1_ref, bhh1_ref,
             iw2_ref, wih2_ref, whh2_ref, bih2_ref, bhh2_ref,
             row_ref, col_ref, degp_ref,
             w2_ref, xw1t_ref, p1_ref, pc_ref, dinv_ref, dinv2_ref):
    deg = jnp.sum(degp_ref[...], axis=0, keepdims=True) + 1.0
    dinv = jnp.where(deg > 0, lax.rsqrt(jnp.maximum(deg, 1e-12)), 0.0)
    dinv_ref[...] = dinv
    dinv2_ref[...] = dinv * dinv
    w1 = _gru(iw1_ref[...], wih1_ref[...], whh1_ref[...], bih1_ref[...],
              bhh1_ref[...])
    w2_ref[...] = _gru(iw2_ref[...], wih2_ref[...], whh2_ref[...],
                       bih2_ref[...], bhh2_ref[...])
    xw1t = lax.dot_general(w1, x_ref[...], (((0,), (1,)), ((), ())),
                           preferred_element_type=jnp.float32)
    xw1t_ref[...] = xw1t
    p1_ref[...] = _pack_pairs(xw1t)
    r = row_ref[...].astype(jnp.uint32)
    c = col_ref[...].astype(jnp.uint32)
    pc_ref[...] = lax.bitcast_convert_type((r << 16) | c, jnp.int32)


def _comb_mm_tc(e1_ref, xwt_ref, d2_ref, w_ref, xw2t_ref, p2_ref):
    h = jnp.maximum(e1_ref[...] + d2_ref[...] * xwt_ref[...], 0.0)
    xw2t = _mmT(w_ref[...], h)
    xw2t_ref[...] = xw2t
    p2_ref[...] = _pack_pairs(xw2t)


def _final_tc(e2_ref, xwt_ref, d2_ref, out_ref):
    out_ref[...] = jnp.transpose(
        jnp.maximum(e2_ref[...] + d2_ref[...] * xwt_ref[...], 0.0), (1, 0))


def kernel(x, edge_index, edge_weight, init_w1, wih1, whh1, bih1, bhh1,
           init_w2, wih2, whh2, bih2, bhh2):
    row = edge_index[0]
    col = edge_index[1]

    f32 = jnp.float32
    deg_p = _deg_sc(col, edge_weight)

    w2, xw1t, p1, pc2d, dinv, dinv2 = pl.pallas_call(
        _prep_tc,
        out_shape=(jax.ShapeDtypeStruct((D, D), f32),
                   jax.ShapeDtypeStruct((D, N), f32),
                   jax.ShapeDtypeStruct((D // 2, N), jnp.int32),
                   jax.ShapeDtypeStruct((E // D, D), jnp.int32),
                   jax.ShapeDtypeStruct((1, N), f32),
                   jax.ShapeDtypeStruct((1, N), f32)),
    )(x, init_w1, wih1, whh1, bih1.reshape(1, 3 * D), bhh1.reshape(1, 3 * D),
      init_w2, wih2, whh2, bih2.reshape(1, 3 * D), bhh2.reshape(1, 3 * D),
      row.reshape(E // D, D), col.reshape(E // D, D), deg_p)
    pc = pc2d.reshape(E)

    nrm = _norm_sc(row, col, edge_weight, dinv.reshape(N))

    e1p = _agg_sc(p1, pc, nrm)

    xw2t, p2 = pl.pallas_call(
        _comb_mm_tc,
        out_shape=(jax.ShapeDtypeStruct((D, N), f32),
                   jax.ShapeDtypeStruct((D // 2, N), jnp.int32)),
    )(e1p, xw1t, dinv2, w2)

    e2p = _agg_sc(p2, pc, nrm)

    out = pl.pallas_call(
        _final_tc,
        out_shape=jax.ShapeDtypeStruct((N, D), f32),
    )(e2p, xw2t, dinv2)
    return out
